# baseline (speedup 1.0000x reference)
import jax
import jax.numpy as jnp
from jax.experimental import pallas as pl
from jax.experimental.pallas import tpu as pltpu

_DIM = 768
_NH = 12
_HD = 64
_E = 8
_HID = 256
_SH = 768
_EPS = 1e-05
_TL = 256
_TQ = 1024
_L = 2048
_BF = jnp.bfloat16
_LOG2E = 1.4426950408889634
_HP = 2 * _HD


def _rms(x, w):
    return x * jax.lax.rsqrt(jnp.mean(x * x, axis=-1, keepdims=True) + _EPS) * w


def _dot_t(a, b):
    return jax.lax.dot_general(a, b, (((1,), (1,)), ((), ())),
                               preferred_element_type=jnp.float32)


def _dot(a, b):
    return jax.lax.dot_general(a, b, (((1,), (0,)), ((), ())),
                               preferred_element_type=jnp.float32)


def _qkv_body(x_ref, n1_ref, wq_ref, wk_ref, wv_ref, cos_ref, sin_ref,
              q_ref, k_ref, v_ref, wq_s, wk_s, wv_s, wqr_s, wkr_s):
    @pl.when(pl.program_id(0) == 0)
    def _cast():
        wq_s[...] = wq_ref[...].astype(_BF)
        wk_s[...] = wk_ref[...].astype(_BF)
        wv_s[...] = wv_ref[...].astype(_BF)
        d = _HD // 2
        for hh in range(_NH):
            r0 = hh * _HD
            wqr_s[r0:r0 + d, :] = -wq_ref[r0 + d:r0 + _HD, :].astype(_BF)
            wqr_s[r0 + d:r0 + _HD, :] = wq_ref[r0:r0 + d, :].astype(_BF)
            wkr_s[r0:r0 + d, :] = -wk_ref[r0 + d:r0 + _HD, :].astype(_BF)
            wkr_s[r0 + d:r0 + _HD, :] = wk_ref[r0:r0 + d, :].astype(_BF)

    xn = _rms(x_ref[...], n1_ref[...]).astype(_BF)
    cos = cos_ref[...]
    sin = sin_ref[...]
    q = _dot_t(xn, wq_s[...]).astype(_BF)
    qr = _dot_t(xn, wqr_s[...]).astype(_BF)
    q_ref[...] = q * cos + qr * sin
    k = _dot_t(xn, wk_s[...]).astype(_BF)
    kr = _dot_t(xn, wkr_s[...]).astype(_BF)
    k_ref[...] = k * cos + kr * sin
    v_ref[...] = _dot_t(xn, wv_s[...]).astype(_BF)


def _attn_body(q_ref, k_ref, v_ref, o_ref):
    q2 = q_ref[...] * _BF(_HD ** -0.5 * _LOG2E)
    k2 = k_ref[...]
    v2 = v_ref[...]
    outs = []
    for jh in range(2):
        sl = slice(jh * _HD, (jh + 1) * _HD)
        s = _dot_t(q2[:, sl], k2[:, sl]).astype(_BF)
        m = jnp.max(s, axis=-1, keepdims=True)
        p = jax.lax.exp2(s - m)
        l = jnp.sum(p, axis=-1, keepdims=True).astype(jnp.float32)
        o = _dot(p, v2[:, sl])
        outs.append((o * (1.0 / l)).astype(_BF))
    o_ref[...] = jnp.concatenate(outs, axis=-1)


import functools
from jax import lax
from jax.experimental.pallas import tpu_sc as plsc

_NC = 2
_NS = 16
_NW = _NC * _NS
_CH = _L // _NW
_VL = 16


def _post_body(a_ref, x_ref, wo_ref, n2_ref, gw_ref, h_ref, hn_ref, lg_ref,
               wo_s):
    @pl.when(pl.program_id(0) == 0)
    def _cast():
        wo_s[...] = wo_ref[...].astype(_BF)

    h = x_ref[...] + _dot_t(a_ref[...], wo_s[...])
    h_ref[...] = h
    hn = _rms(h, n2_ref[...])
    hnb = hn.astype(_BF)
    hn_ref[...] = hnb
    lg_ref[...] = _dot_t(hnb, gw_ref[...].astype(_BF))


def _router_sc_body(lt_hbm, wd_hbm, in_s, out_s):
    wid = lax.axis_index("s") * _NC + lax.axis_index("c")
    base = wid * _CH
    for e in range(_E):
        pltpu.sync_copy(lt_hbm.at[e, pl.ds(base, _CH)], in_s.at[e])
    for j in range(_CH // _VL):
        sl = pl.ds(j * _VL, _VL)
        vals = [in_s[e, sl] for e in range(_E)]
        m1 = vals[0]
        a1 = jnp.zeros((_VL,), jnp.float32)
        for e in range(1, _E):
            upd = vals[e] > m1
            m1 = jnp.where(upd, vals[e], m1)
            a1 = jnp.where(upd, jnp.float32(e), a1)
        m2 = None
        a2 = None
        for e in range(_E):
            veff = jnp.where(a1 == jnp.float32(e),
                             jnp.full((_VL,), -1e30, jnp.float32), vals[e])
            if m2 is None:
                m2 = veff
                a2 = jnp.zeros((_VL,), jnp.float32)
            else:
                upd = veff > m2
                m2 = jnp.where(upd, veff, m2)
                a2 = jnp.where(upd, jnp.float32(e), a2)
        w1 = 1.0 / (1.0 + jnp.exp(m2 - m1))
        w2 = 1.0 - w1
        zero = jnp.zeros((_VL,), jnp.float32)
        for e in range(_E):
            fe = jnp.float32(e)
            out_s[e, sl] = (jnp.where(a1 == fe, w1, zero) +
                            jnp.where(a2 == fe, w2, zero))
    for e in range(_E):
        pltpu.sync_copy(out_s.at[e], wd_hbm.at[e, pl.ds(base, _CH)])


def _moe_body(h_ref, hn_ref, wdt_ref, s1_ref, s2_ref, f1_ref, s3_ref,
              f2_ref, o_ref, s1_s, s2_s, f1_s, s3_s, f2_s):
    @pl.when(pl.program_id(0) == 0)
    def _cast():
        s1_s[...] = s1_ref[...].astype(_BF)
        s2_s[...] = s2_ref[...].astype(_BF)
        f1_s[...] = f1_ref[...].astype(_BF)
        s3_s[...] = s3_ref[...].astype(_BF)
        f2_s[...] = f2_ref[...].astype(_BF)

    h = h_ref[...]
    hnb = hn_ref[...]
    wdt = wdt_ref[...].astype(_BF)
    lane_e = jax.lax.broadcasted_iota(jnp.int32, (_E, _E * _HID), 1) // _HID
    row_e = jax.lax.broadcasted_iota(jnp.int32, (_E, _E * _HID), 0)
    rmat = (lane_e == row_e).astype(_BF)
    wexp = jax.lax.dot_general(wdt, rmat, (((0,), (0,)), ((), ())),
                               preferred_element_type=jnp.float32).astype(_BF)
    g = (jax.nn.silu(_dot_t(hnb, s1_s[...])) *
         _dot_t(hnb, s2_s[...])).astype(_BF)
    acc = h + _dot_t(g, s3_s[...])
    he = (jax.nn.silu(_dot_t(hnb, f1_s[...])) * wexp).astype(_BF)
    for e in range(_E):
        acc = acc + _dot_t(he[:, e * _HID:(e + 1) * _HID], f2_s[e])
    o_ref[...] = acc


def kernel(x, wq, wk, wv, wo, norm1_w, norm2_w, gate_w, fc1_w, fc2_w,
           sh1_w, sh2_w, sh3_w):
    B, L, D = x.shape
    xf = x.reshape(L, D)
    NQ = L // _TL
    n1 = norm1_w.reshape(1, D)
    n2 = norm2_w.reshape(1, D)
    fc1c = fc1_w.reshape(_E * _HID, D)

    inv = 1.0 / (10000.0 ** (jnp.arange(0, 64, 2, dtype=jnp.float32) / 64))
    t = jnp.arange(L, dtype=jnp.float32)
    freqs = jnp.outer(t, inv)
    emb = jnp.concatenate([freqs, freqs], axis=-1)
    cos = jnp.tile(jnp.cos(emb), (1, _NH)).astype(_BF)
    sin = jnp.tile(jnp.sin(emb), (1, _NH)).astype(_BF)

    q, k, v = pl.pallas_call(
        _qkv_body,
        grid=(NQ,),
        in_specs=[
            pl.BlockSpec((_TL, D), lambda i: (i, 0)),
            pl.BlockSpec((1, D), lambda i: (0, 0)),
            pl.BlockSpec((D, D), lambda i: (0, 0)),
            pl.BlockSpec((D, D), lambda i: (0, 0)),
            pl.BlockSpec((D, D), lambda i: (0, 0)),
            pl.BlockSpec((_TL, D), lambda i: (i, 0)),
            pl.BlockSpec((_TL, D), lambda i: (i, 0)),
        ],
        out_specs=[pl.BlockSpec((_TL, D), lambda i: (i, 0))] * 3,
        out_shape=[jax.ShapeDtypeStruct((L, D), _BF)] * 3,
        scratch_shapes=[pltpu.VMEM((D, D), _BF)] * 5,
    )(xf, n1, wq, wk, wv, cos, sin)

    HP = _HP
    a = pl.pallas_call(
        _attn_body,
        grid=(_NH // 2, L // _TQ),
        in_specs=[
            pl.BlockSpec((_TQ, HP), lambda h, i: (i, h)),
            pl.BlockSpec((L, HP), lambda h, i: (0, h)),
            pl.BlockSpec((L, HP), lambda h, i: (0, h)),
        ],
        out_specs=pl.BlockSpec((_TQ, HP), lambda h, i: (i, h)),
        out_shape=jax.ShapeDtypeStruct((L, D), _BF),
    )(q, k, v)

    h, hnb, logits = pl.pallas_call(
        _post_body,
        grid=(NQ,),
        in_specs=[
            pl.BlockSpec((_TL, D), lambda i: (i, 0)),
            pl.BlockSpec((_TL, D), lambda i: (i, 0)),
            pl.BlockSpec((D, D), lambda i: (0, 0)),
            pl.BlockSpec((1, D), lambda i: (0, 0)),
            pl.BlockSpec((_E, D), lambda i: (0, 0)),
        ],
        out_specs=[
            pl.BlockSpec((_TL, D), lambda i: (i, 0)),
            pl.BlockSpec((_TL, D), lambda i: (i, 0)),
            pl.BlockSpec((_TL, _E), lambda i: (i, 0)),
        ],
        out_shape=[
            jax.ShapeDtypeStruct((L, D), jnp.float32),
            jax.ShapeDtypeStruct((L, D), _BF),
            jax.ShapeDtypeStruct((L, _E), jnp.float32),
        ],
        scratch_shapes=[pltpu.VMEM((D, D), _BF)],
    )(a, xf, wo, n2, gate_w)

    lt = logits.T

    mesh = plsc.VectorSubcoreMesh(core_axis_name="c", subcore_axis_name="s")
    wdt = functools.partial(
        pl.kernel, mesh=mesh,
        out_type=jax.ShapeDtypeStruct((_E, _L), jnp.float32),
        scratch_types=[
            pltpu.VMEM((_E, _CH), jnp.float32),
            pltpu.VMEM((_E, _CH), jnp.float32),
        ],
    )(_router_sc_body)(lt)

    out = pl.pallas_call(
        _moe_body,
        grid=(NQ,),
        in_specs=[
            pl.BlockSpec((_TL, D), lambda i: (i, 0)),
            pl.BlockSpec((_TL, D), lambda i: (i, 0)),
            pl.BlockSpec((_E, _TL), lambda i: (0, i)),
            pl.BlockSpec((_SH, D), lambda i: (0, 0)),
            pl.BlockSpec((_SH, D), lambda i: (0, 0)),
            pl.BlockSpec((_E * _HID, D), lambda i: (0, 0)),
            pl.BlockSpec((D, _SH), lambda i: (0, 0)),
            pl.BlockSpec((_E, D, _HID), lambda i: (0, 0, 0)),
        ],
        out_specs=pl.BlockSpec((_TL, D), lambda i: (i, 0)),
        out_shape=jax.ShapeDtypeStruct((L, D), jnp.float32),
        scratch_shapes=[
            pltpu.VMEM((_SH, D), _BF),
            pltpu.VMEM((_SH, D), _BF),
            pltpu.VMEM((_E * _HID, D), _BF),
            pltpu.VMEM((D, _SH), _BF),
            pltpu.VMEM((_E, D, _HID), _BF),
        ],
    )(h, hnb, wdt, sh1_w, sh2_w, fc1c, sh3_w, fc2_w)

    return out.reshape(B, L, D)


# speedup vs baseline: 1.1398x; 1.1398x over previous
import jax
import jax.numpy as jnp
from jax.experimental import pallas as pl
from jax.experimental.pallas import tpu as pltpu

_DIM = 768
_NH = 12
_HD = 64
_E = 8
_HID = 256
_SH = 768
_EPS = 1e-05
_TL = 256
_TQ = 1024
_L = 2048
_BF = jnp.bfloat16
_LOG2E = 1.4426950408889634
_HP = 2 * _HD


def _rms(x, w):
    return x * jax.lax.rsqrt(jnp.mean(x * x, axis=-1, keepdims=True) + _EPS) * w


def _dot_t(a, b):
    return jax.lax.dot_general(a, b, (((1,), (1,)), ((), ())),
                               preferred_element_type=jnp.float32)


def _dot(a, b):
    return jax.lax.dot_general(a, b, (((1,), (0,)), ((), ())),
                               preferred_element_type=jnp.float32)


def _qkv_body(x_ref, n1_ref, wq_ref, wk_ref, wv_ref, cos_ref, sin_ref,
              q_ref, k_ref, v_ref, wq_s, wk_s, wv_s, wqr_s, wkr_s):
    @pl.when(pl.program_id(0) == 0)
    def _cast():
        wq_s[...] = wq_ref[...].astype(_BF)
        wk_s[...] = wk_ref[...].astype(_BF)
        wv_s[...] = wv_ref[...].astype(_BF)
        d = _HD // 2
        for hh in range(_NH):
            r0 = hh * _HD
            wqr_s[r0:r0 + d, :] = -wq_ref[r0 + d:r0 + _HD, :].astype(_BF)
            wqr_s[r0 + d:r0 + _HD, :] = wq_ref[r0:r0 + d, :].astype(_BF)
            wkr_s[r0:r0 + d, :] = -wk_ref[r0 + d:r0 + _HD, :].astype(_BF)
            wkr_s[r0 + d:r0 + _HD, :] = wk_ref[r0:r0 + d, :].astype(_BF)

    xn = _rms(x_ref[...], n1_ref[...]).astype(_BF)
    cos = cos_ref[...]
    sin = sin_ref[...]
    q = _dot_t(xn, wq_s[...]).astype(_BF)
    qr = _dot_t(xn, wqr_s[...]).astype(_BF)
    q_ref[...] = q * cos + qr * sin
    k = _dot_t(xn, wk_s[...]).astype(_BF)
    kr = _dot_t(xn, wkr_s[...]).astype(_BF)
    k_ref[...] = k * cos + kr * sin
    v_ref[...] = _dot_t(xn, wv_s[...]).astype(_BF)


def _attn_body(q_ref, k_ref, v_ref, o_ref):
    q2 = q_ref[...] * _BF(_HD ** -0.5 * _LOG2E)
    k2 = k_ref[...]
    v2 = v_ref[...]
    outs = []
    for jh in range(2):
        sl = slice(jh * _HD, (jh + 1) * _HD)
        s = _dot_t(q2[:, sl], k2[:, sl]).astype(_BF)
        m = jnp.max(s, axis=-1, keepdims=True)
        p = jax.lax.exp2(s - m)
        l = jnp.sum(p, axis=-1, keepdims=True).astype(jnp.float32)
        o = _dot(p, v2[:, sl])
        outs.append((o * (1.0 / l)).astype(_BF))
    o_ref[...] = jnp.concatenate(outs, axis=-1)


def _block2_body(a_ref, x_ref, wo_ref, n2_ref, gw_ref, s1_ref, s2_ref,
                 f1_ref, s3_ref, f2_ref, o_ref,
                 wo_s, s1_s, s2_s, f1_s, s3_s, f2_s):
    @pl.when(pl.program_id(0) == 0)
    def _cast():
        wo_s[...] = wo_ref[...].astype(_BF)
        s1_s[...] = s1_ref[...].astype(_BF)
        s2_s[...] = s2_ref[...].astype(_BF)
        f1_s[...] = f1_ref[...].astype(_BF)
        s3_s[...] = s3_ref[...].astype(_BF)
        for e in range(_E):
            f2_s[e * _HID:(e + 1) * _HID, :] = jnp.swapaxes(
                f2_ref[e], 0, 1).astype(_BF)

    h = x_ref[...] + _dot_t(a_ref[...], wo_s[...])
    hn = _rms(h, n2_ref[...])
    hnb = hn.astype(_BF)
    logits = _dot_t(hnb, gw_ref[...].astype(_BF))
    idx = jax.lax.broadcasted_iota(jnp.int32, logits.shape, 1)
    m1 = jnp.max(logits, axis=-1, keepdims=True)
    a1 = jnp.min(jnp.where(logits == m1, idx, _E), axis=-1, keepdims=True)
    oh1 = idx == a1
    masked = jnp.where(oh1, -jnp.inf, logits)
    m2 = jnp.max(masked, axis=-1, keepdims=True)
    a2 = jnp.min(jnp.where(masked == m2, idx, _E), axis=-1, keepdims=True)
    oh2 = idx == a2
    w1 = jax.lax.logistic(m1 - m2)
    wd = (jnp.where(oh1, w1, 0.0) + jnp.where(oh2, 1.0 - w1, 0.0)).astype(_BF)
    lane_e = jax.lax.broadcasted_iota(jnp.int32, (_E, _E * _HID), 1) // _HID
    row_e = jax.lax.broadcasted_iota(jnp.int32, (_E, _E * _HID), 0)
    rmat = (lane_e == row_e).astype(_BF)
    wexp = _dot(wd, rmat).astype(_BF)
    g = (jax.nn.silu(_dot_t(hnb, s1_s[...])) *
         _dot_t(hnb, s2_s[...])).astype(_BF)
    acc = h + _dot_t(g, s3_s[...])
    he = (jax.nn.silu(_dot_t(hnb, f1_s[...])) * wexp).astype(_BF)
    o_ref[...] = acc + _dot(he, f2_s[...])


def kernel(x, wq, wk, wv, wo, norm1_w, norm2_w, gate_w, fc1_w, fc2_w,
           sh1_w, sh2_w, sh3_w):
    B, L, D = x.shape
    xf = x.reshape(L, D)
    NQ = L // _TL
    n1 = norm1_w.reshape(1, D)
    n2 = norm2_w.reshape(1, D)
    fc1c = fc1_w.reshape(_E * _HID, D)

    inv = 1.0 / (10000.0 ** (jnp.arange(0, _HD, 2, dtype=jnp.float32) / _HD))
    t = jnp.arange(L, dtype=jnp.float32)
    freqs = jnp.outer(t, inv)
    emb = jnp.concatenate([freqs, freqs], axis=-1)
    cos = jnp.tile(jnp.cos(emb), (1, _NH)).astype(_BF)
    sin = jnp.tile(jnp.sin(emb), (1, _NH)).astype(_BF)

    q, k, v = pl.pallas_call(
        _qkv_body,
        grid=(NQ,),
        in_specs=[
            pl.BlockSpec((_TL, D), lambda i: (i, 0)),
            pl.BlockSpec((1, D), lambda i: (0, 0)),
            pl.BlockSpec((D, D), lambda i: (0, 0)),
            pl.BlockSpec((D, D), lambda i: (0, 0)),
            pl.BlockSpec((D, D), lambda i: (0, 0)),
            pl.BlockSpec((_TL, D), lambda i: (i, 0)),
            pl.BlockSpec((_TL, D), lambda i: (i, 0)),
        ],
        out_specs=[pl.BlockSpec((_TL, D), lambda i: (i, 0))] * 3,
        out_shape=[jax.ShapeDtypeStruct((L, D), _BF)] * 3,
        scratch_shapes=[pltpu.VMEM((D, D), _BF)] * 5,
    )(xf, n1, wq, wk, wv, cos, sin)

    a = pl.pallas_call(
        _attn_body,
        grid=(_NH // 2, L // _TQ),
        in_specs=[
            pl.BlockSpec((_TQ, _HP), lambda h, i: (i, h)),
            pl.BlockSpec((L, _HP), lambda h, i: (0, h)),
            pl.BlockSpec((L, _HP), lambda h, i: (0, h)),
        ],
        out_specs=pl.BlockSpec((_TQ, _HP), lambda h, i: (i, h)),
        out_shape=jax.ShapeDtypeStruct((L, D), _BF),
    )(q, k, v)

    out = pl.pallas_call(
        _block2_body,
        grid=(NQ,),
        in_specs=[
            pl.BlockSpec((_TL, D), lambda i: (i, 0)),
            pl.BlockSpec((_TL, D), lambda i: (i, 0)),
            pl.BlockSpec((D, D), lambda i: (0, 0)),
            pl.BlockSpec((1, D), lambda i: (0, 0)),
            pl.BlockSpec((_E, D), lambda i: (0, 0)),
            pl.BlockSpec((_SH, D), lambda i: (0, 0)),
            pl.BlockSpec((_SH, D), lambda i: (0, 0)),
            pl.BlockSpec((_E * _HID, D), lambda i: (0, 0)),
            pl.BlockSpec((D, _SH), lambda i: (0, 0)),
            pl.BlockSpec((_E, D, _HID), lambda i: (0, 0, 0)),
        ],
        out_specs=pl.BlockSpec((_TL, D), lambda i: (i, 0)),
        out_shape=jax.ShapeDtypeStruct((L, D), jnp.float32),
        scratch_shapes=[
            pltpu.VMEM((D, D), _BF),
            pltpu.VMEM((_SH, D), _BF),
            pltpu.VMEM((_SH, D), _BF),
            pltpu.VMEM((_E * _HID, D), _BF),
            pltpu.VMEM((D, _SH), _BF),
            pltpu.VMEM((_E * _HID, D), _BF),
        ],
    )(a, xf, wo, n2, gate_w, sh1_w, sh2_w, fc1c, sh3_w, fc2_w)

    return out.reshape(B, L, D)


# speedup vs baseline: 1.1460x; 1.0054x over previous
import jax
import jax.numpy as jnp
from jax.experimental import pallas as pl
from jax.experimental.pallas import tpu as pltpu

_DIM = 768
_NH = 12
_HD = 64
_E = 8
_HID = 256
_SH = 768
_EPS = 1e-05
_TL = 256
_TQ = 1024
_L = 2048
_BF = jnp.bfloat16
_LOG2E = 1.4426950408889634
_HP = 2 * _HD


def _rms(x, w):
    return x * jax.lax.rsqrt(jnp.mean(x * x, axis=-1, keepdims=True) + _EPS) * w


def _dot_t(a, b):
    return jax.lax.dot_general(a, b, (((1,), (1,)), ((), ())),
                               preferred_element_type=jnp.float32)


def _dot(a, b):
    return jax.lax.dot_general(a, b, (((1,), (0,)), ((), ())),
                               preferred_element_type=jnp.float32)


def _qkv_body(x_ref, n1_ref, wq_ref, wk_ref, wv_ref, cos_ref, sin_ref,
              q_ref, k_ref, v_ref, wq_s, wk_s, wv_s, wqr_s, wkr_s):
    @pl.when(pl.program_id(0) == 0)
    def _cast():
        wq_s[...] = wq_ref[...].astype(_BF)
        wk_s[...] = wk_ref[...].astype(_BF)
        wv_s[...] = wv_ref[...].astype(_BF)
        d = _HD // 2
        for hh in range(_NH):
            r0 = hh * _HD
            wqr_s[r0:r0 + d, :] = -wq_ref[r0 + d:r0 + _HD, :].astype(_BF)
            wqr_s[r0 + d:r0 + _HD, :] = wq_ref[r0:r0 + d, :].astype(_BF)
            wkr_s[r0:r0 + d, :] = -wk_ref[r0 + d:r0 + _HD, :].astype(_BF)
            wkr_s[r0 + d:r0 + _HD, :] = wk_ref[r0:r0 + d, :].astype(_BF)

    xn = _rms(x_ref[...], n1_ref[...]).astype(_BF)
    cos = cos_ref[...]
    sin = sin_ref[...]
    q = _dot_t(xn, wq_s[...]).astype(_BF)
    qr = _dot_t(xn, wqr_s[...]).astype(_BF)
    q_ref[...] = q * cos + qr * sin
    k = _dot_t(xn, wk_s[...]).astype(_BF)
    kr = _dot_t(xn, wkr_s[...]).astype(_BF)
    k_ref[...] = k * cos + kr * sin
    v_ref[...] = _dot_t(xn, wv_s[...]).astype(_BF)


def _attn_body(q_ref, k_ref, v_ref, o_ref):
    q2 = q_ref[...] * _BF(_HD ** -0.5 * _LOG2E)
    k2 = k_ref[...]
    v2 = v_ref[...]
    outs = []
    for jh in range(2):
        sl = slice(jh * _HD, (jh + 1) * _HD)
        s = _dot_t(q2[:, sl], k2[:, sl]).astype(_BF)
        m = jnp.max(s, axis=-1, keepdims=True)
        p = jax.lax.exp2(s - m)
        l = jnp.sum(p, axis=-1, keepdims=True).astype(jnp.float32)
        o = _dot(p, v2[:, sl])
        outs.append((o * (1.0 / l)).astype(_BF))
    o_ref[...] = jnp.concatenate(outs, axis=-1)


def _block2_body(a_ref, x_ref, wo_ref, n2_ref, gw_ref, s1_ref, s2_ref,
                 f1_ref, s3_ref, f2_ref, o_ref,
                 wo_s, s1_s, s2_s, f1_s, s3_s, f2_s):
    @pl.when(pl.program_id(0) == 0)
    def _cast():
        wo_s[...] = wo_ref[...].astype(_BF)
        s1_s[...] = s1_ref[...].astype(_BF)
        s2_s[...] = s2_ref[...].astype(_BF)
        f1_s[...] = f1_ref[...].astype(_BF)
        s3_s[...] = s3_ref[...].astype(_BF)
        f2_s[...] = f2_ref[...].astype(_BF)

    h = x_ref[...] + _dot_t(a_ref[...], wo_s[...])
    hn = _rms(h, n2_ref[...])
    hnb = hn.astype(_BF)
    logits = _dot_t(hnb, gw_ref[...].astype(_BF))
    idx = jax.lax.broadcasted_iota(jnp.int32, logits.shape, 1)
    m1 = jnp.max(logits, axis=-1, keepdims=True)
    a1 = jnp.min(jnp.where(logits == m1, idx, _E), axis=-1, keepdims=True)
    oh1 = idx == a1
    masked = jnp.where(oh1, -jnp.inf, logits)
    m2 = jnp.max(masked, axis=-1, keepdims=True)
    a2 = jnp.min(jnp.where(masked == m2, idx, _E), axis=-1, keepdims=True)
    oh2 = idx == a2
    w1 = jax.lax.logistic(m1 - m2)
    wd = (jnp.where(oh1, w1, 0.0) + jnp.where(oh2, 1.0 - w1, 0.0)).astype(_BF)
    lane_e = jax.lax.broadcasted_iota(jnp.int32, (_E, _E * _HID), 1) // _HID
    row_e = jax.lax.broadcasted_iota(jnp.int32, (_E, _E * _HID), 0)
    rmat = (lane_e == row_e).astype(_BF)
    wexp = _dot(wd, rmat).astype(_BF)
    g = (jax.nn.silu(_dot_t(hnb, s1_s[...])) *
         _dot_t(hnb, s2_s[...])).astype(_BF)
    acc = h + _dot_t(g, s3_s[...])
    he = (jax.nn.silu(_dot_t(hnb, f1_s[...])) * wexp).astype(_BF)
    for e in range(_E):
        acc = acc + _dot_t(he[:, e * _HID:(e + 1) * _HID], f2_s[e])
    o_ref[...] = acc


def kernel(x, wq, wk, wv, wo, norm1_w, norm2_w, gate_w, fc1_w, fc2_w,
           sh1_w, sh2_w, sh3_w):
    B, L, D = x.shape
    xf = x.reshape(L, D)
    NQ = L // _TL
    n1 = norm1_w.reshape(1, D)
    n2 = norm2_w.reshape(1, D)
    fc1c = fc1_w.reshape(_E * _HID, D)

    inv = 1.0 / (10000.0 ** (jnp.arange(0, _HD, 2, dtype=jnp.float32) / _HD))
    t = jnp.arange(L, dtype=jnp.float32)
    freqs = jnp.outer(t, inv)
    emb = jnp.concatenate([freqs, freqs], axis=-1)
    cos = jnp.tile(jnp.cos(emb), (1, _NH)).astype(_BF)
    sin = jnp.tile(jnp.sin(emb), (1, _NH)).astype(_BF)

    q, k, v = pl.pallas_call(
        _qkv_body,
        grid=(NQ,),
        in_specs=[
            pl.BlockSpec((_TL, D), lambda i: (i, 0)),
            pl.BlockSpec((1, D), lambda i: (0, 0)),
            pl.BlockSpec((D, D), lambda i: (0, 0)),
            pl.BlockSpec((D, D), lambda i: (0, 0)),
            pl.BlockSpec((D, D), lambda i: (0, 0)),
            pl.BlockSpec((_TL, D), lambda i: (i, 0)),
            pl.BlockSpec((_TL, D), lambda i: (i, 0)),
        ],
        out_specs=[pl.BlockSpec((_TL, D), lambda i: (i, 0))] * 3,
        out_shape=[jax.ShapeDtypeStruct((L, D), _BF)] * 3,
        scratch_shapes=[pltpu.VMEM((D, D), _BF)] * 5,
    )(xf, n1, wq, wk, wv, cos, sin)

    a = pl.pallas_call(
        _attn_body,
        grid=(_NH // 2, L // _TQ),
        in_specs=[
            pl.BlockSpec((_TQ, _HP), lambda h, i: (i, h)),
            pl.BlockSpec((L, _HP), lambda h, i: (0, h)),
            pl.BlockSpec((L, _HP), lambda h, i: (0, h)),
        ],
        out_specs=pl.BlockSpec((_TQ, _HP), lambda h, i: (i, h)),
        out_shape=jax.ShapeDtypeStruct((L, D), _BF),
    )(q, k, v)

    out = pl.pallas_call(
        _block2_body,
        grid=(NQ,),
        in_specs=[
            pl.BlockSpec((_TL, D), lambda i: (i, 0)),
            pl.BlockSpec((_TL, D), lambda i: (i, 0)),
            pl.BlockSpec((D, D), lambda i: (0, 0)),
            pl.BlockSpec((1, D), lambda i: (0, 0)),
            pl.BlockSpec((_E, D), lambda i: (0, 0)),
            pl.BlockSpec((_SH, D), lambda i: (0, 0)),
            pl.BlockSpec((_SH, D), lambda i: (0, 0)),
            pl.BlockSpec((_E * _HID, D), lambda i: (0, 0)),
            pl.BlockSpec((D, _SH), lambda i: (0, 0)),
            pl.BlockSpec((_E, D, _HID), lambda i: (0, 0, 0)),
        ],
        out_specs=pl.BlockSpec((_TL, D), lambda i: (i, 0)),
        out_shape=jax.ShapeDtypeStruct((L, D), jnp.float32),
        scratch_shapes=[
            pltpu.VMEM((D, D), _BF),
            pltpu.VMEM((_SH, D), _BF),
            pltpu.VMEM((_SH, D), _BF),
            pltpu.VMEM((_E * _HID, D), _BF),
            pltpu.VMEM((D, _SH), _BF),
            pltpu.VMEM((_E, D, _HID), _BF),
        ],
    )(a, xf, wo, n2, gate_w, sh1_w, sh2_w, fc1c, sh3_w, fc2_w)

    return out.reshape(B, L, D)
